# Initial kernel scaffold; baseline (speedup 1.0000x reference)
import jax
import jax.numpy as jnp
from jax.experimental import pallas as pl


def kernel(x, edge_index, batch, mask, W1, b1, W2, b2, Wf1, bf1, Wf3, bf3):
    raise NotImplementedError("write your pallas kernel here")


# speedup vs baseline: 27.5485x; 27.5485x over previous
import functools

import jax
import jax.numpy as jnp
from jax import lax
from jax.experimental import pallas as pl
from jax.experimental.pallas import tpu as pltpu
from jax.experimental.pallas import tpu_sc as plsc

NC = 2
NS = 16
NW = NC * NS
NP = 10240
RPW = NP // NS
G = 64
C = 128


def _leaky(v):
    return jnp.where(v >= 0, v, 0.01 * v)


def _make_deg_kernel(E):
    EPW = E // NW
    nfull = EPW // C
    rem = EPW - nfull * C
    mesh = plsc.VectorSubcoreMesh(core_axis_name="c", subcore_axis_name="s")

    @functools.partial(
        pl.kernel,
        out_type=jax.ShapeDtypeStruct((NC, NP), jnp.float32),
        mesh=mesh,
        scratch_types=[
            pltpu.VMEM((C,), jnp.int32),
            pltpu.VMEM((rem,), jnp.int32),
            pltpu.VMEM((C,), jnp.float32),
            pltpu.VMEM((rem,), jnp.float32),
            pltpu.VMEM((RPW,), jnp.float32),
            pltpu.VMEM_SHARED((NP,), jnp.float32),
        ],
    )
    def deg_k(dst_hbm, out_hbm, idx_v, idxr_v, ones_v, onesr_v, z_v, acc_sh):
        c = lax.axis_index("c")
        s = lax.axis_index("s")
        base = (c * NS + s) * EPW

        def _zfill(i, _):
            z_v[pl.ds(i * 16, 16)] = jnp.zeros((16,), jnp.float32)
            return 0

        lax.fori_loop(0, RPW // 16, _zfill, 0)
        for i in range(C // 16):
            ones_v[pl.ds(i * 16, 16)] = jnp.ones((16,), jnp.float32)
        onesr_v[...] = jnp.ones((rem,), jnp.float32)
        pltpu.sync_copy(z_v, acc_sh.at[pl.ds(s * RPW, RPW)])
        plsc.subcore_barrier()

        def body(i, _):
            pltpu.sync_copy(dst_hbm.at[pl.ds(base + i * C, C)], idx_v)
            pltpu.sync_copy(ones_v, acc_sh.at[idx_v], add=True)
            return 0

        lax.fori_loop(0, nfull, body, 0)
        if rem:
            pltpu.sync_copy(dst_hbm.at[pl.ds(base + nfull * C, rem)], idxr_v)
            pltpu.sync_copy(onesr_v, acc_sh.at[idxr_v], add=True)
        plsc.subcore_barrier()
        pltpu.sync_copy(acc_sh.at[pl.ds(s * RPW, RPW)],
                        out_hbm.at[c, pl.ds(s * RPW, RPW)])

    return deg_k


def _make_agg_kernel(E, D):
    EPW = E // NW
    nfull = EPW // C
    npair = nfull // 2
    rem = EPW - nfull * C
    mesh = plsc.VectorSubcoreMesh(core_axis_name="c", subcore_axis_name="s")

    @functools.partial(
        pl.kernel,
        out_type=jax.ShapeDtypeStruct((NC, NP, D), jnp.float32),
        mesh=mesh,
        scratch_types=[
            pltpu.VMEM((2, C), jnp.int32),
            pltpu.VMEM((2, C), jnp.int32),
            pltpu.VMEM((2, C, D), jnp.float32),
            pltpu.VMEM((rem,), jnp.int32),
            pltpu.VMEM((rem,), jnp.int32),
            pltpu.VMEM((rem, D), jnp.float32),
            pltpu.SemaphoreType.DMA,
            pltpu.SemaphoreType.DMA,
            pltpu.VMEM_SHARED((NP, D), jnp.float32),
        ],
        compiler_params=pltpu.CompilerParams(use_tc_tiling_on_sc=False),
    )
    def agg_k(hs_hbm, src_hbm, dst_hbm, zero_hbm, out_hbm,
              sb, db, rows, sbr, dbr, rowsr, sem0, sem1, acc_sh):
        c = lax.axis_index("c")
        s = lax.axis_index("s")
        base = (c * NS + s) * EPW
        pltpu.sync_copy(zero_hbm.at[pl.ds(s * RPW, RPW)],
                        acc_sh.at[pl.ds(s * RPW, RPW)])
        plsc.subcore_barrier()

        def body(p, _):
            o = base + (2 * p) * C
            pltpu.sync_copy(src_hbm.at[pl.ds(o, C)], sb.at[0])
            pltpu.sync_copy(src_hbm.at[pl.ds(o + C, C)], sb.at[1])
            d0 = pltpu.async_copy(hs_hbm.at[sb.at[0]], rows.at[0], sem0)
            d1 = pltpu.async_copy(hs_hbm.at[sb.at[1]], rows.at[1], sem1)
            pltpu.sync_copy(dst_hbm.at[pl.ds(o, C)], db.at[0])
            pltpu.sync_copy(dst_hbm.at[pl.ds(o + C, C)], db.at[1])
            d0.wait()
            pltpu.sync_copy(rows.at[0], acc_sh.at[db.at[0]], add=True)
            d1.wait()
            pltpu.sync_copy(rows.at[1], acc_sh.at[db.at[1]], add=True)
            return 0

        lax.fori_loop(0, npair, body, 0)
        if rem:
            pltpu.sync_copy(src_hbm.at[pl.ds(base + nfull * C, rem)], sbr)
            pltpu.sync_copy(dst_hbm.at[pl.ds(base + nfull * C, rem)], dbr)
            pltpu.async_copy(hs_hbm.at[sbr], rowsr, sem0).wait()
            pltpu.sync_copy(rowsr, acc_sh.at[dbr], add=True)
        plsc.subcore_barrier()
        pltpu.sync_copy(acc_sh.at[pl.ds(s * RPW, RPW)],
                        out_hbm.at[c, pl.ds(s * RPW, RPW)])

    return agg_k


def _k1_body(x_ref, w1_ref, degp_ref, hs_ref, dis_ref):
    d = degp_ref[:, 0] + degp_ref[:, 1] + 1.0
    dis = lax.rsqrt(d)[:, None]
    h = jnp.dot(x_ref[...], w1_ref[...], preferred_element_type=jnp.float32)
    hs_ref[...] = h * dis
    dis_ref[...] = dis


def _k3_body(acc_ref, hs1_ref, dis_ref, b1_ref, w2_ref, hs2_ref):
    dis = dis_ref[...]
    a = acc_ref[0] + acc_ref[1] + hs1_ref[...]
    o = _leaky(a * dis + b1_ref[...])
    hs2_ref[...] = jnp.dot(o, w2_ref[...],
                           preferred_element_type=jnp.float32) * dis


def _k5_body(acc_ref, hs2_ref, dis_ref, b2_ref, batch_ref,
             wf1_ref, bf1_ref, wf3_ref, bf3_ref, out_ref, sums, cnts):
    i = pl.program_id(0)

    @pl.when(i == 0)
    def _():
        sums[...] = jnp.zeros_like(sums)
        cnts[...] = jnp.zeros_like(cnts)

    a = acc_ref[0] + acc_ref[1] + hs2_ref[...]
    o = _leaky(a * dis_ref[...] + b2_ref[...])
    gids = lax.broadcasted_iota(jnp.int32, (1, G), 1)
    oh = (batch_ref[...] == gids).astype(jnp.float32)
    sums[...] += lax.dot_general(oh, o, (((0,), (0,)), ((), ())),
                                 preferred_element_type=jnp.float32)
    ones_col = jnp.ones((oh.shape[0], 1), jnp.float32)
    cnts[...] += lax.dot_general(oh, ones_col, (((0,), (0,)), ((), ())),
                                 preferred_element_type=jnp.float32)

    @pl.when(i == pl.num_programs(0) - 1)
    def _():
        pooled = sums[...] / jnp.maximum(cnts[...], 1.0)
        z = _leaky(jnp.dot(pooled, wf1_ref[...],
                           preferred_element_type=jnp.float32) + bf1_ref[...])
        out_ref[...] = jnp.dot(z, wf3_ref[...],
                               preferred_element_type=jnp.float32) + bf3_ref[...]


def kernel(x, edge_index, batch, mask, W1, b1, W2, b2, Wf1, bf1, Wf3, bf3):
    N, F_IN = x.shape
    E = edge_index.shape[1]
    D1 = W1.shape[1]
    D2 = W2.shape[1]
    NCLS = Wf3.shape[1]
    BN = 2000
    grid = N // BN

    src = edge_index[0]
    dst = edge_index[1]
    degp = _make_deg_kernel(E)(dst)
    degp_t = degp.T

    hs1, dis = pl.pallas_call(
        _k1_body,
        grid=(grid,),
        in_specs=[
            pl.BlockSpec((BN, F_IN), lambda i: (i, 0)),
            pl.BlockSpec((F_IN, D1), lambda i: (0, 0)),
            pl.BlockSpec((BN, NC), lambda i: (i, 0)),
        ],
        out_specs=[
            pl.BlockSpec((BN, D1), lambda i: (i, 0)),
            pl.BlockSpec((BN, 1), lambda i: (i, 0)),
        ],
        out_shape=[
            jax.ShapeDtypeStruct((N, D1), jnp.float32),
            jax.ShapeDtypeStruct((N, 1), jnp.float32),
        ],
    )(x, W1, degp_t)

    zeros1 = jnp.zeros((NP, D1), jnp.float32)
    acc1 = _make_agg_kernel(E, D1)(hs1, src, dst, zeros1)

    hs2 = pl.pallas_call(
        _k3_body,
        grid=(grid,),
        in_specs=[
            pl.BlockSpec((NC, BN, D1), lambda i: (0, i, 0)),
            pl.BlockSpec((BN, D1), lambda i: (i, 0)),
            pl.BlockSpec((BN, 1), lambda i: (i, 0)),
            pl.BlockSpec((1, D1), lambda i: (0, 0)),
            pl.BlockSpec((D1, D2), lambda i: (0, 0)),
        ],
        out_specs=pl.BlockSpec((BN, D2), lambda i: (i, 0)),
        out_shape=jax.ShapeDtypeStruct((N, D2), jnp.float32),
    )(acc1, hs1, dis, b1[None, :], W2)

    zeros2 = jnp.zeros((NP, D2), jnp.float32)
    acc2 = _make_agg_kernel(E, D2)(hs2, src, dst, zeros2)

    out = pl.pallas_call(
        _k5_body,
        grid=(grid,),
        in_specs=[
            pl.BlockSpec((NC, BN, D2), lambda i: (0, i, 0)),
            pl.BlockSpec((BN, D2), lambda i: (i, 0)),
            pl.BlockSpec((BN, 1), lambda i: (i, 0)),
            pl.BlockSpec((1, D2), lambda i: (0, 0)),
            pl.BlockSpec((BN, 1), lambda i: (i, 0)),
            pl.BlockSpec((D2, Wf1.shape[1]), lambda i: (0, 0)),
            pl.BlockSpec((1, Wf1.shape[1]), lambda i: (0, 0)),
            pl.BlockSpec((Wf1.shape[1], NCLS), lambda i: (0, 0)),
            pl.BlockSpec((1, NCLS), lambda i: (0, 0)),
        ],
        out_specs=pl.BlockSpec((G, NCLS), lambda i: (0, 0)),
        out_shape=jax.ShapeDtypeStruct((G, NCLS), jnp.float32),
        scratch_shapes=[
            pltpu.VMEM((G, D2), jnp.float32),
            pltpu.VMEM((G, 1), jnp.float32),
        ],
    )(acc2, hs2, dis, b2[None, :], batch[:, None],
      Wf1, bf1[None, :], Wf3, bf3[None, :])

    return out
